# baseline (speedup 1.0000x reference)
import functools

import jax
import jax.numpy as jnp
from jax import lax
from jax.experimental import pallas as pl
from jax.experimental.pallas import tpu as pltpu
from jax.experimental.pallas import tpu_sc as plsc


@functools.lru_cache(maxsize=None)
def _make_gather(BATCH, FIELDS, D, NC, NS):
    NW = NC * NS
    BB = 128
    n_blocks = BATCH * FIELDS // BB
    bpw = n_blocks // NW
    RW = 128
    PACK = RW // D
    mesh = plsc.VectorSubcoreMesh(core_axis_name="c", subcore_axis_name="s")

    @functools.partial(
        pl.kernel,
        mesh=mesh,
        out_type=jax.ShapeDtypeStruct((FIELDS, D, BATCH), jnp.float32),
        scratch_types=[
            pltpu.VMEM((2, BB), jnp.int32),
            pltpu.VMEM((2, BB), jnp.int32),
            pltpu.VMEM((2, BB), jnp.int32),
            pltpu.VMEM((2, BB, RW), jnp.float32),
            pltpu.VMEM((2, D, BB), jnp.float32),
            pltpu.SemaphoreType.DMA((2,)),
            pltpu.SemaphoreType.DMA((2,)),
        ],
        compiler_params=pltpu.CompilerParams(
            use_tc_tiling_on_sc=True, needs_layout_passes=False),
    )
    def gather_kernel(idx_hbm, table_hbm, out_hbm, idxv, rowv, offv, gbuf, obuf,
                      gsem, ssem):
        wid = lax.axis_index("s") * NC + lax.axis_index("c")
        blk0 = wid * bpw
        iotas = [lax.iota(jnp.int32, 16) + 16 * i for i in range(BB // 16)]

        def prime(g, p):
            blk = blk0 + g
            j = blk // (BATCH // BB)
            cb = blk % (BATCH // BB)
            fb = j * BATCH + cb * BB
            pltpu.sync_copy(idx_hbm.at[pl.ds(fb, BB)], idxv.at[p])
            for i in range(BB // 16):
                v = idxv[p, pl.ds(16 * i, 16)]
                rowv[p, pl.ds(16 * i, 16)] = v >> 2
                offv[p, pl.ds(16 * i, 16)] = (v & 3) << 5
            pltpu.async_copy(table_hbm.at[rowv.at[p]], gbuf.at[p], gsem.at[p])

        def wait_gather(p):
            pltpu.make_async_copy(
                table_hbm.at[rowv.at[p]], gbuf.at[p], gsem.at[p]).wait()

        def assemble_store(g, p, first):
            if not first:
                pltpu.make_async_copy(
                    obuf.at[p], out_hbm.at[0, :, pl.ds(0, BB)], ssem.at[p]
                ).wait()
            bases = [offv[p, pl.ds(16 * i, 16)] for i in range(BB // 16)]
            for d in range(D):
                for i in range(BB // 16):
                    vec = plsc.load_gather(
                        gbuf.at[p], [iotas[i], bases[i] + d])
                    obuf[p, d, pl.ds(16 * i, 16)] = vec
            blk = blk0 + g
            j = blk // (BATCH // BB)
            cb = blk % (BATCH // BB)
            pltpu.async_copy(
                obuf.at[p], out_hbm.at[j, :, pl.ds(cb * BB, BB)], ssem.at[p])

        prime(0, 0)
        prime(1, 1)

        def body(g2, carry):
            for u in range(2):
                g = 2 * g2 + u
                wait_gather(u)
                assemble_store(g, u, first=False)
                prime(g + 2, u)
            return carry

        wait_gather(0)
        assemble_store(0, 0, first=True)
        prime(2, 0)
        wait_gather(1)
        assemble_store(1, 1, first=True)
        prime(3, 1)
        lax.fori_loop(1, bpw // 2 - 1, body, 0)
        for u in range(2):
            g = bpw - 2 + u
            wait_gather(u)
            assemble_store(g, u, first=False)
        for p in range(2):
            pltpu.make_async_copy(
                obuf.at[p], out_hbm.at[0, :, pl.ds(0, BB)], ssem.at[p]).wait()

    return gather_kernel


def kernel(input, embedding_matrix):
    BATCH, FIELDS = input.shape
    V, D = embedding_matrix.shape
    info = plsc.get_sparse_core_info()
    NC, NS = info.num_cores, info.num_subcores
    idx_fb = input.T.reshape(BATCH * FIELDS).astype(jnp.int32)
    table4 = embedding_matrix.reshape(V * D // 128, 128)
    out = _make_gather(BATCH, FIELDS, D, NC, NS)(idx_fb, table4)
    return jnp.transpose(out, (2, 0, 1))
